# baseline (speedup 1.0000x reference)
import functools

import jax
import jax.numpy as jnp
from jax import lax
from jax.experimental import pallas as pl
from jax.experimental.pallas import tpu as pltpu
from jax.experimental.pallas import tpu_sc as plsc

BATCH = 4096
SEQ = 200
D = 64
SCALE = 8.0

_info = plsc.get_sparse_core_info()
NC, NS, L = _info.num_cores, _info.num_subcores, _info.num_lanes
NW = NC * NS

ROWS_W = BATCH // NW
CLEN = (104, 96)
COFF = (0, 104)
CPR = 2
N_CHUNKS = ROWS_W * CPR
NB = 4
LA = 2


def _emb_body(x_hbm, table_hbm, out_hbm, idx_slab, rows, *sems):
    gsem, ssem = sems[:NB], sems[NB:]
    wid = lax.axis_index("s") * NC + lax.axis_index("c")
    bbase = wid * ROWS_W

    pltpu.sync_copy(x_hbm.at[pl.ds(bbase, ROWS_W)], idx_slab)

    def gather_ops(j, b, h):
        br = j // CPR
        src = table_hbm.at[idx_slab.at[br, pl.ds(COFF[h], CLEN[h])]]
        dst = rows.at[b, pl.ds(0, CLEN[h])]
        return src, dst

    def scatter_ops(j, b, h):
        br = j // CPR
        src = rows.at[b, pl.ds(0, CLEN[h])]
        dst = out_hbm.at[bbase + br, pl.ds(COFF[h], CLEN[h])]
        return src, dst

    def start_gather(j, b, h):
        src, dst = gather_ops(j, b, h)
        pltpu.async_copy(src, dst, gsem[b])

    def wait_gather(j, b, h):
        src, dst = gather_ops(j, b, h)
        pltpu.make_async_copy(src, dst, gsem[b]).wait()

    def start_scatter(j, b, h):
        src, dst = scatter_ops(j, b, h)
        pltpu.async_copy(src, dst, ssem[b])

    def wait_scatter(j, b, h):
        src, dst = scatter_ops(j, b, h)
        pltpu.make_async_copy(src, dst, ssem[b]).wait()

    for b in range(LA):
        start_gather(b, b, b % CPR)

    def group(g, carry):
        for b in range(NB):
            j = g * NB + b
            h = b % CPR
            jf = j + LA
            bf = (b + LA) % NB
            hf = bf % CPR

            @pl.when(jnp.logical_and(jf < N_CHUNKS, jf >= NB))
            def _():
                wait_scatter(jf - NB, bf, hf)

            @pl.when(jf < N_CHUNKS)
            def _():
                start_gather(jf, bf, hf)

            wait_gather(j, b, h)

            def row(i, c2):
                r = i * 2
                for rr in range(2):
                    for c in range(D // L):
                        rows[b, r + rr, pl.ds(c * L, L)] = (
                            rows[b, r + rr, pl.ds(c * L, L)] * SCALE)
                return c2

            lax.fori_loop(0, CLEN[h] // 2, row, 0)
            start_scatter(j, b, h)
        return carry

    lax.fori_loop(0, N_CHUNKS // NB, group, 0)

    for b in range(NB):
        wait_scatter(N_CHUNKS - NB + b, b, b % CPR)


_emb_kernel = functools.partial(
    pl.kernel,
    out_type=jax.ShapeDtypeStruct((BATCH, SEQ, D), jnp.float32),
    mesh=plsc.VectorSubcoreMesh(core_axis_name="c", subcore_axis_name="s"),
    compiler_params=pltpu.CompilerParams(use_tc_tiling_on_sc=False),
    scratch_types=(
        [pltpu.VMEM((ROWS_W, SEQ), jnp.int32),
         pltpu.VMEM((NB, CLEN[0], D), jnp.float32)]
        + [pltpu.SemaphoreType.DMA] * (2 * NB)
    ),
)(_emb_body)


def kernel(x, table):
    return _emb_kernel(x, table)
